# Initial kernel scaffold; baseline (speedup 1.0000x reference)
import functools

import jax
import jax.numpy as jnp
from jax import lax
from jax.experimental import pallas as pl
from jax.experimental.pallas import tpu as pltpu
from jax.experimental.pallas import tpu_sc as plsc

N = 10000
NPAD = 10240
D = 128
E = 320000
NC, NS = 2, 16
NW = NC * NS
CH = 128
GPT = 80
EPAD = NW * GPT * CH
ROWS_PER_TILE = NPAD // NS

_mesh = plsc.VectorSubcoreMesh(
    core_axis_name="c", subcore_axis_name="s", num_cores=NC, num_subcores=NS)


@functools.partial(
    pl.kernel,
    out_type=(jax.ShapeDtypeStruct((NC, NPAD, 16), jnp.float32),
              jax.ShapeDtypeStruct((NC, NPAD, 16), jnp.float32)),
    mesh=_mesh,
    scratch_types=[
        pltpu.VMEM((GPT, CH), jnp.int32),
        pltpu.VMEM((GPT, CH), jnp.int32),
        pltpu.VMEM((CH, 16), jnp.float32),
        pltpu.VMEM_SHARED((NPAD, 16), jnp.float32),
        pltpu.VMEM_SHARED((NPAD, 16), jnp.float32),
    ],
)
def _degrees(srcp, dstp, ones_hbm, zeros_hbm, out_s, out_d,
             src_v, dst_v, ones_v, acc_s, acc_d):
    c = lax.axis_index("c")
    s = lax.axis_index("s")
    wid = c * NS + s
    pltpu.sync_copy(srcp.at[wid], src_v)
    pltpu.sync_copy(dstp.at[wid], dst_v)
    pltpu.sync_copy(ones_hbm, ones_v)
    pltpu.sync_copy(zeros_hbm, acc_s.at[pl.ds(s * ROWS_PER_TILE, ROWS_PER_TILE)])
    pltpu.sync_copy(zeros_hbm, acc_d.at[pl.ds(s * ROWS_PER_TILE, ROWS_PER_TILE)])
    plsc.subcore_barrier()

    def body(g, carry):
        pltpu.sync_copy(ones_v, acc_s.at[src_v.at[g]], add=True)
        pltpu.sync_copy(ones_v, acc_d.at[dst_v.at[g]], add=True)
        return carry

    lax.fori_loop(0, GPT, body, 0)
    plsc.subcore_barrier()
    sl = pl.ds(s * ROWS_PER_TILE, ROWS_PER_TILE)
    pltpu.sync_copy(acc_s.at[sl], out_s.at[c, sl])
    pltpu.sync_copy(acc_d.at[sl], out_d.at[c, sl])


@functools.partial(
    pl.kernel,
    out_type=jax.ShapeDtypeStruct((NC, NPAD, D), jnp.float32),
    mesh=_mesh,
    scratch_types=[
        pltpu.VMEM((GPT, CH), jnp.int32),
        pltpu.VMEM((GPT, CH), jnp.int32),
        pltpu.VMEM((2, CH, D), jnp.float32),
        pltpu.VMEM_SHARED((NPAD, D), jnp.float32),
        pltpu.SemaphoreType.DMA,
        pltpu.SemaphoreType.DMA,
    ],
)
def _spmm(table, srcp, dstp, zeros_hbm, out,
          src_v, dst_v, rows, acc, sem0, sem1):
    c = lax.axis_index("c")
    s = lax.axis_index("s")
    wid = c * NS + s
    pltpu.sync_copy(srcp.at[wid], src_v)
    pltpu.sync_copy(dstp.at[wid], dst_v)
    pltpu.sync_copy(zeros_hbm, acc.at[pl.ds(s * ROWS_PER_TILE, ROWS_PER_TILE)])
    plsc.subcore_barrier()

    sems = (sem0, sem1)
    pltpu.async_copy(table.at[src_v.at[0]], rows.at[0], sem0)
    pltpu.async_copy(table.at[src_v.at[1]], rows.at[1], sem1)

    def body(g2, carry):
        g = g2 * 2
        for b in range(2):
            gb = g + b
            pltpu.make_async_copy(table.at[src_v.at[gb]], rows.at[b], sems[b]).wait()
            pltpu.sync_copy(rows.at[b], acc.at[dst_v.at[gb]], add=True)
            nxt = gb + 2

            @pl.when(nxt < GPT)
            def _start():
                pltpu.async_copy(table.at[src_v.at[nxt]], rows.at[b], sems[b])
        return carry

    lax.fori_loop(0, GPT // 2, body, 0)
    plsc.subcore_barrier()
    sl = pl.ds(s * ROWS_PER_TILE, ROWS_PER_TILE)
    pltpu.sync_copy(acc.at[sl], out.at[c, sl])


def _norm_from(deg2):
    deg = deg2[0, :, 0] + deg2[1, :, 0]
    return jnp.where(deg > 0.0, lax.rsqrt(jnp.maximum(deg, 1.0)), 0.0)


def _tc1_body(x_ref, degs_ref, w_ref, out_ref):
    ns = _norm_from(degs_ref[...])
    xs = x_ref[...] * ns[:, None]
    out_ref[...] = jnp.dot(xs, w_ref[...], preferred_element_type=jnp.float32)


def _tc2_body(agg_ref, degs_ref, degd_ref, b_ref, w_ref, out_ref):
    agg = agg_ref[0] + agg_ref[1]
    nd = _norm_from(degd_ref[...])
    h = jnp.maximum(agg * nd[:, None] + b_ref[...], 0.0)
    ns = _norm_from(degs_ref[...])
    out_ref[...] = jnp.dot(h * ns[:, None], w_ref[...],
                           preferred_element_type=jnp.float32)


def _tc3_body(agg_ref, degd_ref, b_ref, wc_ref, bc_ref, out_ref):
    agg = agg_ref[0] + agg_ref[1]
    nd = _norm_from(degd_ref[...])
    h = jnp.maximum(agg * nd[:, None] + b_ref[...], 0.0)
    hg = jnp.sum(h[:N, :], axis=0, keepdims=True) * (1.0 / N)
    out_ref[...] = jnp.dot(hg, wc_ref[...],
                           preferred_element_type=jnp.float32) + bc_ref[...]


def kernel(x, edge_index, W1, b1, W2, b2, Wc, bc):
    f32 = jnp.float32
    src = edge_index[0].astype(jnp.int32)
    dst = edge_index[1].astype(jnp.int32)
    npad_rows = NPAD - N
    padv = N + (jnp.arange(EPAD - E, dtype=jnp.int32) % npad_rows)
    srcp = jnp.concatenate([src, padv]).reshape(NW, GPT, CH)
    dstp = jnp.concatenate([dst, padv]).reshape(NW, GPT, CH)
    xp = jnp.concatenate([x, jnp.zeros((npad_rows, D), f32)], axis=0)

    ones16 = jnp.ones((CH, 16), f32)
    z16 = jnp.zeros((ROWS_PER_TILE, 16), f32)
    z128 = jnp.zeros((ROWS_PER_TILE, D), f32)

    deg_s, deg_d = _degrees(srcp, dstp, ones16, z16)

    hw1 = pl.pallas_call(
        _tc1_body,
        out_shape=jax.ShapeDtypeStruct((NPAD, D), f32),
    )(xp, deg_s, W1)

    agg1 = _spmm(hw1, srcp, dstp, z128)

    hw2 = pl.pallas_call(
        _tc2_body,
        out_shape=jax.ShapeDtypeStruct((NPAD, D), f32),
    )(agg1, deg_s, deg_d, b1.reshape(1, D), W2)

    agg2 = _spmm(hw2, srcp, dstp, z128)

    wcp = jnp.zeros((D, D), f32).at[:, :Wc.shape[1]].set(Wc)
    bcp = jnp.zeros((1, D), f32).at[0, :bc.shape[0]].set(bc)
    outp = pl.pallas_call(
        _tc3_body,
        out_shape=jax.ShapeDtypeStruct((1, D), f32),
    )(agg2, deg_d, b2.reshape(1, D), wcp, bcp)
    return outp[:, :Wc.shape[1]]


# speedup vs baseline: 8.5551x; 8.5551x over previous
import functools

import jax
import jax.numpy as jnp
from jax import lax
from jax.experimental import pallas as pl
from jax.experimental.pallas import tpu as pltpu
from jax.experimental.pallas import tpu_sc as plsc

N = 10000
NPAD = 10240
D = 128
DH = 64
E = 320000
NC, NS = 2, 16
NW = NC * NS
CH = 128
GPT_DEG = 80
GPT = 160
EPAD = NS * GPT * CH
RPT = NPAD // NS


@functools.cache
def _make_degrees():
  mesh = plsc.VectorSubcoreMesh(
      core_axis_name="c", subcore_axis_name="s",
      num_cores=NC, num_subcores=NS)
  return functools.partial(
      pl.kernel,
      out_type=(jax.ShapeDtypeStruct((NW, NPAD), jnp.int32),
                jax.ShapeDtypeStruct((NW, NPAD), jnp.int32)),
      mesh=mesh,
      scratch_types=[
          pltpu.VMEM((GPT_DEG, CH), jnp.int32),
          pltpu.VMEM((GPT_DEG, CH), jnp.int32),
          pltpu.VMEM((NPAD,), jnp.int32),
          pltpu.VMEM((NPAD,), jnp.int32),
      ],
      compiler_params=pltpu.CompilerParams(needs_layout_passes=False),
  )(_degrees_body)


def _degrees_body(srcp, dstp, zeros_hbm, out_s, out_d,
                  src_v, dst_v, hist_s, hist_d):
  c = lax.axis_index("c")
  s = lax.axis_index("s")
  wid = c * NS + s
  pltpu.sync_copy(srcp.at[wid], src_v)
  pltpu.sync_copy(dstp.at[wid], dst_v)
  pltpu.sync_copy(zeros_hbm, hist_s)
  pltpu.sync_copy(zeros_hbm, hist_d)

  def body(g, carry):
    for j in range(CH // 16):
      sl = pl.ds(j * 16, 16)
      for idx_v, hist in ((src_v, hist_s), (dst_v, hist_d)):
        idx = idx_v.at[g][sl]
        cnt, last = plsc.scan_count(idx)
        plsc.addupdate_scatter(hist, [idx], cnt, mask=last)
    return carry

  lax.fori_loop(0, GPT_DEG, body, 0)
  pltpu.sync_copy(hist_s, out_s.at[wid])
  pltpu.sync_copy(hist_d, out_d.at[wid])


NH = NPAD // NC
NDUMP = 128
NACC = NH + NDUMP
ZR = NACC // NS
WR = NH // NS


@functools.cache
def _make_spmm():
  mesh = plsc.VectorSubcoreMesh(
      core_axis_name="c", subcore_axis_name="s",
      num_cores=NC, num_subcores=NS)
  return functools.partial(
      pl.kernel,
      out_type=jax.ShapeDtypeStruct((NPAD, D), jnp.float32),
      mesh=mesh,
      scratch_types=[
          pltpu.VMEM((GPT, CH), jnp.int32),
          pltpu.VMEM((GPT, CH), jnp.int32),
          pltpu.VMEM((1, CH), jnp.int32),
          pltpu.VMEM((2, CH, D), jnp.float32),
          pltpu.VMEM_SHARED((NACC, D), jnp.float32),
          pltpu.SemaphoreType.DMA,
          pltpu.SemaphoreType.DMA,
      ],
  )(_spmm_body)


def _spmm_body(tbl, srcp, dstp, zeros_hbm, out,
               src_v, dst_v, dloc_v, rows, acc, sem0, sem1):
  c = lax.axis_index("c")
  s = lax.axis_index("s")
  base = c * NH
  pltpu.sync_copy(srcp.at[s], src_v)
  pltpu.sync_copy(dstp.at[s], dst_v)
  pltpu.sync_copy(zeros_hbm, acc.at[pl.ds(s * ZR, ZR)])
  plsc.subcore_barrier()
  sems = (sem0, sem1)

  def localize(g):
    for j in range(CH // 16):
      sl = pl.ds(j * 16, 16)
      dv = dst_v.at[g][sl]
      lv = dv - base
      valid = (lv >= 0) & (lv < NH)
      dump = NH + j * 16 + lax.iota(jnp.int32, 16)
      dloc_v.at[0][sl] = jnp.where(valid, lv, dump)

  pltpu.async_copy(tbl.at[src_v.at[0]], rows.at[0], sem0)
  pltpu.async_copy(tbl.at[src_v.at[1]], rows.at[1], sem1)

  def body(g2, carry):
    g = g2 * 2
    for b in range(2):
      gb = g + b
      localize(gb)
      pltpu.make_async_copy(tbl.at[src_v.at[gb]], rows.at[b], sems[b]).wait()
      pltpu.sync_copy(rows.at[b], acc.at[dloc_v.at[0]], add=True)
      nxt = gb + 2

      @pl.when(nxt < GPT)
      def _start():
        pltpu.async_copy(tbl.at[src_v.at[nxt]], rows.at[b], sems[b])
    return carry

  lax.fori_loop(0, GPT // 2, body, 0)
  plsc.subcore_barrier()
  pltpu.sync_copy(acc.at[pl.ds(s * WR, WR)],
                  out.at[pl.ds(base + s * WR, WR)])


def _norm_from(hist):
  deg = jnp.sum(hist, axis=0).astype(jnp.float32)
  return jnp.where(deg > 0.0, lax.rsqrt(jnp.maximum(deg, 1.0)), 0.0)


def _tc1_body(x_ref, degs_ref, w_ref, out_ref):
  ns = _norm_from(degs_ref[...])
  xs = x_ref[...] * ns[:, None]
  out_ref[...] = jnp.dot(xs, w_ref[...], preferred_element_type=jnp.float32)


def _tc2_body(agg_ref, degs_ref, degd_ref, b_ref, w_ref, out_ref):
  nd = _norm_from(degd_ref[...])
  h = jnp.maximum(agg_ref[...] * nd[:, None] + b_ref[...], 0.0)
  ns = _norm_from(degs_ref[...])
  out_ref[...] = jnp.dot(h * ns[:, None], w_ref[...],
                         preferred_element_type=jnp.float32)


def _tc3_body(agg_ref, degd_ref, b_ref, wc_ref, bc_ref, out_ref):
  nd = _norm_from(degd_ref[...])
  h = jnp.maximum(agg_ref[...] * nd[:, None] + b_ref[...], 0.0)
  hg = jnp.sum(h[:N, :], axis=0, keepdims=True) * (1.0 / N)
  out_ref[...] = jnp.dot(hg, wc_ref[...],
                         preferred_element_type=jnp.float32) + bc_ref[...]


def kernel(x, edge_index, W1, b1, W2, b2, Wc, bc):
  f32 = jnp.float32
  src = edge_index[0].astype(jnp.int32)
  dst = edge_index[1].astype(jnp.int32)
  npad_rows = NPAD - N
  padv = N + (jnp.arange(EPAD - E, dtype=jnp.int32) % npad_rows)
  src_flat = jnp.concatenate([src, padv])
  dst_flat = jnp.concatenate([dst, padv])
  srcp32 = src_flat.reshape(NW, GPT_DEG, CH)
  dstp32 = dst_flat.reshape(NW, GPT_DEG, CH)
  srcp16 = src_flat.reshape(NS, GPT, CH)
  dstp16 = dst_flat.reshape(NS, GPT, CH)
  xp = jnp.concatenate([x, jnp.zeros((npad_rows, D), f32)], axis=0)

  zi = jnp.zeros((NPAD,), jnp.int32)
  zacc = jnp.zeros((ZR, D), f32)

  deg_s, deg_d = _make_degrees()(srcp32, dstp32, zi)

  hw1 = pl.pallas_call(
      _tc1_body,
      out_shape=jax.ShapeDtypeStruct((NPAD, D), f32),
  )(xp, deg_s, W1)

  agg1 = _make_spmm()(hw1, srcp16, dstp16, zacc)

  hw2 = pl.pallas_call(
      _tc2_body,
      out_shape=jax.ShapeDtypeStruct((NPAD, D), f32),
  )(agg1, deg_s, deg_d, b1.reshape(1, D), W2)

  agg2 = _make_spmm()(hw2, srcp16, dstp16, zacc)

  wcp = jnp.zeros((D, D), f32).at[:, :Wc.shape[1]].set(Wc)
  bcp = jnp.zeros((1, D), f32).at[0, :bc.shape[0]].set(bc)
  outp = pl.pallas_call(
      _tc3_body,
      out_shape=jax.ShapeDtypeStruct((1, D), f32),
  )(agg2, deg_d, b2.reshape(1, D), wcp, bcp)
  return outp[:, :Wc.shape[1]]


# speedup vs baseline: 11.5756x; 1.3531x over previous
import functools

import jax
import jax.numpy as jnp
from jax import lax
from jax.experimental import pallas as pl
from jax.experimental.pallas import tpu as pltpu
from jax.experimental.pallas import tpu_sc as plsc

N = 10000
NPAD = 10240
D = 128
DH = 64
E = 320000
NC, NS = 2, 16
NW = NC * NS
CH = 128
GPT_DEG = 80
EPAD = NW * GPT_DEG * CH
RPT = NPAD // NS


@functools.cache
def _make_degrees():
  mesh = plsc.VectorSubcoreMesh(
      core_axis_name="c", subcore_axis_name="s",
      num_cores=NC, num_subcores=NS)
  return functools.partial(
      pl.kernel,
      out_type=(jax.ShapeDtypeStruct((NW, NPAD), jnp.int32),
                jax.ShapeDtypeStruct((NW, NPAD), jnp.int32)),
      mesh=mesh,
      scratch_types=[
          pltpu.VMEM((GPT_DEG, CH), jnp.int32),
          pltpu.VMEM((GPT_DEG, CH), jnp.int32),
          pltpu.VMEM((NPAD,), jnp.int32),
          pltpu.VMEM((NPAD,), jnp.int32),
      ],
      compiler_params=pltpu.CompilerParams(needs_layout_passes=False),
  )(_degrees_body)


def _degrees_body(srcp, dstp, zeros_hbm, out_s, out_d,
                  src_v, dst_v, hist_s, hist_d):
  c = lax.axis_index("c")
  s = lax.axis_index("s")
  wid = c * NS + s
  pltpu.sync_copy(srcp.at[wid], src_v)
  pltpu.sync_copy(dstp.at[wid], dst_v)
  pltpu.sync_copy(zeros_hbm, hist_s)
  pltpu.sync_copy(zeros_hbm, hist_d)

  def body(g, carry):
    for j in range(CH // 16):
      sl = pl.ds(j * 16, 16)
      for idx_v, hist in ((src_v, hist_s), (dst_v, hist_d)):
        idx = idx_v.at[g][sl]
        cnt, last = plsc.scan_count(idx)
        plsc.addupdate_scatter(hist, [idx], cnt, mask=last)
    return carry

  lax.fori_loop(0, GPT_DEG, body, 0)
  pltpu.sync_copy(hist_s, out_s.at[wid])
  pltpu.sync_copy(hist_d, out_d.at[wid])


NB = 4
NH = NPAD // NC
NDUMP = 128
NACC = NH + NDUMP
ZR = NACC // NS
WR = NH // NS
LSEG = GPT_DEG
LCAP = LSEG * CH + 16


@functools.cache
def _make_route():
  mesh = plsc.VectorSubcoreMesh(
      core_axis_name="c", subcore_axis_name="s",
      num_cores=NC, num_subcores=NS)
  return functools.partial(
      pl.kernel,
      out_type=(jax.ShapeDtypeStruct((NW, NC, LSEG * CH), jnp.int32),
                jax.ShapeDtypeStruct((NW, NC, LSEG * CH), jnp.int32),
                jax.ShapeDtypeStruct((NW, 16), jnp.int32)),
      mesh=mesh,
      scratch_types=[
          pltpu.VMEM((GPT_DEG, CH), jnp.int32),
          pltpu.VMEM((GPT_DEG, CH), jnp.int32),
          pltpu.VMEM((LCAP,), jnp.int32),
          pltpu.VMEM((LCAP,), jnp.int32),
          pltpu.VMEM((LCAP,), jnp.int32),
          pltpu.VMEM((LCAP,), jnp.int32),
          pltpu.VMEM((16,), jnp.int32),
      ],
      compiler_params=pltpu.CompilerParams(needs_layout_passes=False),
  )(_route_body)


def _route_body(srcp, dstp, padsrc_t, paddst_t, rsrc, rdst, rcnt,
                src_v, dst_v, lsrc0, lsrc1, ldst0, ldst1, cnt_v):
  c = lax.axis_index("c")
  s = lax.axis_index("s")
  wid = c * NS + s
  pltpu.sync_copy(srcp.at[wid], src_v)
  pltpu.sync_copy(dstp.at[wid], dst_v)
  pltpu.sync_copy(padsrc_t, lsrc0)
  pltpu.sync_copy(padsrc_t, lsrc1)
  pltpu.sync_copy(paddst_t, ldst0)
  pltpu.sync_copy(paddst_t, ldst1)

  def body(g, carry):
    cnt0, cnt1 = carry
    for j in range(CH // 16):
      sl = pl.ds(j * 16, 16)
      sv = src_v.at[g][sl]
      dv = dst_v.at[g][sl]
      m0 = dv < NH
      ones0 = jnp.where(m0, 1, 0).astype(jnp.int32)
      pos0 = cnt0 + plsc.cumsum(ones0) - 1
      plsc.store_scatter(lsrc0, [pos0], sv, mask=m0)
      plsc.store_scatter(ldst0, [pos0], dv, mask=m0)
      cnt0 = cnt0 + plsc.all_reduce_population_count(m0)
      m1 = jnp.logical_not(m0)
      ones1 = jnp.where(m1, 1, 0).astype(jnp.int32)
      pos1 = cnt1 + plsc.cumsum(ones1) - 1
      plsc.store_scatter(lsrc1, [pos1], sv, mask=m1)
      plsc.store_scatter(ldst1, [pos1], dv, mask=m1)
      cnt1 = cnt1 + plsc.all_reduce_population_count(m1)
    return cnt0, cnt1

  zero16 = jnp.zeros((16,), jnp.int32)
  cnt0, cnt1 = lax.fori_loop(0, GPT_DEG, body, (zero16, zero16))
  lane = lax.iota(jnp.int32, 16)
  cnt_v[...] = jnp.where(lane == 0, cnt0, jnp.where(lane == 1, cnt1, 0))
  pltpu.sync_copy(cnt_v, rcnt.at[wid])
  pltpu.sync_copy(lsrc0.at[pl.ds(0, LSEG * CH)], rsrc.at[wid, 0])
  pltpu.sync_copy(lsrc1.at[pl.ds(0, LSEG * CH)], rsrc.at[wid, 1])
  pltpu.sync_copy(ldst0.at[pl.ds(0, LSEG * CH)], rdst.at[wid, 0])
  pltpu.sync_copy(ldst1.at[pl.ds(0, LSEG * CH)], rdst.at[wid, 1])


@functools.cache
def _make_spmm():
  mesh = plsc.VectorSubcoreMesh(
      core_axis_name="c", subcore_axis_name="s",
      num_cores=NC, num_subcores=NS)
  return functools.partial(
      pl.kernel,
      out_type=jax.ShapeDtypeStruct((NPAD, D), jnp.float32),
      mesh=mesh,
      scratch_types=[
          pltpu.VMEM((LSEG, CH), jnp.int32),
          pltpu.VMEM((LSEG, CH), jnp.int32),
          pltpu.VMEM((NB, CH), jnp.int32),
          pltpu.VMEM((NB, CH, D), jnp.float32),
          pltpu.VMEM((16,), jnp.int32),
          pltpu.VMEM_SHARED((NACC, D), jnp.float32),
          pltpu.SemaphoreType.DMA,
          pltpu.SemaphoreType.DMA,
      ],
  )(_spmm_body)


def _spmm_body(tbl, rsrc, rdst, rcnt, zeros_hbm, out,
               src_v, dst_v, dloc_v, rows, cnt_v, acc, gsem0, gsem1):
  c = lax.axis_index("c")
  s = lax.axis_index("s")
  base = c * NH
  pltpu.sync_copy(zeros_hbm, acc.at[pl.ds(s * ZR, ZR)])
  plsc.subcore_barrier()
  gsems = (gsem0, gsem1)

  def localize(g, b):
    for j in range(CH // 16):
      sl = pl.ds(j * 16, 16)
      dv = dst_v.at[g][sl]
      lv = dv - base
      valid = (lv >= 0) & (lv < NH)
      dump = NH + j * 16 + lax.iota(jnp.int32, 16)
      dloc_v.at[b][sl] = jnp.where(valid, lv, dump)

  def run_segment(k):
    w = 2 * s + k
    pltpu.sync_copy(rsrc.at[w, c], src_v)
    pltpu.sync_copy(rdst.at[w, c], dst_v)
    pltpu.sync_copy(rcnt.at[w], cnt_v)
    cv = cnt_v[...]
    cnt = jnp.where(c == 0, cv[0], cv[1])
    nchunk = lax.div(cnt + (CH - 1), CH)
    nt = jnp.maximum(lax.div(nchunk + (NB - 1), NB), 1)

    def gather(g, b):
      pltpu.async_copy(tbl.at[src_v.at[g]], rows.at[b], gsems[b % 2])

    gather(0, 0)
    gather(1, 1)

    def body(g4, carry):
      for b in range(NB):
        g = g4 * NB + b
        pltpu.make_async_copy(tbl.at[src_v.at[g]], rows.at[b],
                              gsems[b % 2]).wait()

        @pl.when(g + 2 < nt * NB)
        def _prefetch():
          gather(g + 2, (b + 2) % NB)

        localize(g, b)
        pltpu.sync_copy(rows.at[b], acc.at[dloc_v.at[b]], add=True)
      return carry

    lax.fori_loop(0, nt, body, 0)

  run_segment(0)
  run_segment(1)
  plsc.subcore_barrier()
  pltpu.sync_copy(acc.at[pl.ds(s * WR, WR)],
                  out.at[pl.ds(base + s * WR, WR)])


def _norm_from(hist):
  deg = jnp.sum(hist, axis=0).astype(jnp.float32)
  return jnp.where(deg > 0.0, lax.rsqrt(jnp.maximum(deg, 1.0)), 0.0)


def _tc1_body(x_ref, degs_ref, w_ref, out_ref):
  ns = _norm_from(degs_ref[...])
  xs = x_ref[...] * ns[:, None]
  out_ref[...] = jnp.dot(xs, w_ref[...], preferred_element_type=jnp.float32)


def _tc2_body(agg_ref, degs_ref, degd_ref, b_ref, w_ref, out_ref):
  nd = _norm_from(degd_ref[...])
  h = jnp.maximum(agg_ref[...] * nd[:, None] + b_ref[...], 0.0)
  ns = _norm_from(degs_ref[...])
  out_ref[...] = jnp.dot(h * ns[:, None], w_ref[...],
                         preferred_element_type=jnp.float32)


def _tc3_body(agg_ref, degd_ref, b_ref, wc_ref, bc_ref, out_ref):
  nd = _norm_from(degd_ref[...])
  h = jnp.maximum(agg_ref[...] * nd[:, None] + b_ref[...], 0.0)
  hg = jnp.sum(h[:N, :], axis=0, keepdims=True) * (1.0 / N)
  out_ref[...] = jnp.dot(hg, wc_ref[...],
                         preferred_element_type=jnp.float32) + bc_ref[...]


def kernel(x, edge_index, W1, b1, W2, b2, Wc, bc):
  f32 = jnp.float32
  src = edge_index[0].astype(jnp.int32)
  dst = edge_index[1].astype(jnp.int32)
  npad_rows = NPAD - N
  padv = N + (jnp.arange(EPAD - E, dtype=jnp.int32) % npad_rows)
  src_flat = jnp.concatenate([src, padv])
  dst_flat = jnp.concatenate([dst, padv])
  srcp32 = src_flat.reshape(NW, GPT_DEG, CH)
  dstp32 = dst_flat.reshape(NW, GPT_DEG, CH)
  xp = jnp.concatenate([x, jnp.zeros((npad_rows, D), f32)], axis=0)

  zi = jnp.zeros((NPAD,), jnp.int32)
  zacc = jnp.zeros((ZR, D), f32)
  padsrc_t = N + (jnp.arange(LCAP, dtype=jnp.int32) % npad_rows)
  paddst_t = jnp.full((LCAP,), 2 * NPAD, jnp.int32)

  rsrc, rdst, rcnt = _make_route()(srcp32, dstp32, padsrc_t, paddst_t)
  rsrc4 = rsrc.reshape(NW, NC, LSEG, CH)
  rdst4 = rdst.reshape(NW, NC, LSEG, CH)

  deg_s, deg_d = _make_degrees()(srcp32, dstp32, zi)

  hw1 = pl.pallas_call(
      _tc1_body,
      out_shape=jax.ShapeDtypeStruct((NPAD, D), f32),
  )(xp, deg_s, W1)

  agg1 = _make_spmm()(hw1, rsrc4, rdst4, rcnt, zacc)

  hw2 = pl.pallas_call(
      _tc2_body,
      out_shape=jax.ShapeDtypeStruct((NPAD, D), f32),
  )(agg1, deg_s, deg_d, b1.reshape(1, D), W2)

  agg2 = _make_spmm()(hw2, rsrc4, rdst4, rcnt, zacc)

  wcp = jnp.zeros((D, D), f32).at[:, :Wc.shape[1]].set(Wc)
  bcp = jnp.zeros((1, D), f32).at[0, :bc.shape[0]].set(bc)
  outp = pl.pallas_call(
      _tc3_body,
      out_shape=jax.ShapeDtypeStruct((1, D), f32),
  )(agg2, deg_d, b2.reshape(1, D), wcp, bcp)
  return outp[:, :Wc.shape[1]]


# speedup vs baseline: 12.1075x; 1.0460x over previous
import functools

import jax
import jax.numpy as jnp
from jax import lax
from jax.experimental import pallas as pl
from jax.experimental.pallas import tpu as pltpu
from jax.experimental.pallas import tpu_sc as plsc

N = 10000
NPAD = 10240
D = 128
DH = 64
E = 320000
NC, NS = 2, 16
NW = NC * NS
CH = 128
GPT_DEG = 80
EPAD = NW * GPT_DEG * CH
RPT = NPAD // NS


NB = 4
NH = NPAD // NC
NDUMP = 128
NACC = NH + NDUMP
ZR = NACC // NS
WR = NH // NS
LSEG = GPT_DEG
LCAP = LSEG * CH + 16


@functools.cache
def _make_prep():
  mesh = plsc.VectorSubcoreMesh(
      core_axis_name="c", subcore_axis_name="s",
      num_cores=NC, num_subcores=NS)
  return functools.partial(
      pl.kernel,
      out_type=(jax.ShapeDtypeStruct((NW, NC, LSEG * CH), jnp.int32),
                jax.ShapeDtypeStruct((NW, NC, LSEG * CH), jnp.int32),
                jax.ShapeDtypeStruct((NW, 16), jnp.int32),
                jax.ShapeDtypeStruct((NW, NPAD), jnp.int32),
                jax.ShapeDtypeStruct((NW, NPAD), jnp.int32)),
      mesh=mesh,
      scratch_types=[
          pltpu.VMEM((GPT_DEG, CH), jnp.int32),
          pltpu.VMEM((GPT_DEG, CH), jnp.int32),
          pltpu.VMEM((LCAP,), jnp.int32),
          pltpu.VMEM((LCAP,), jnp.int32),
          pltpu.VMEM((LCAP,), jnp.int32),
          pltpu.VMEM((LCAP,), jnp.int32),
          pltpu.VMEM((16,), jnp.int32),
          pltpu.VMEM((NPAD,), jnp.int32),
          pltpu.VMEM((NPAD,), jnp.int32),
      ],
      compiler_params=pltpu.CompilerParams(needs_layout_passes=False),
  )(_prep_body)


def _prep_body(srcp, dstp, padsrc_t, paddst_t, zeros_hbm,
               rsrc, rdst, rcnt, out_hs, out_hd,
               src_v, dst_v, lsrc0, lsrc1, ldst0, ldst1, cnt_v,
               hist_s, hist_d):
  c = lax.axis_index("c")
  s = lax.axis_index("s")
  wid = c * NS + s
  pltpu.sync_copy(srcp.at[wid], src_v)
  pltpu.sync_copy(dstp.at[wid], dst_v)
  pltpu.sync_copy(padsrc_t, lsrc0)
  pltpu.sync_copy(padsrc_t, lsrc1)
  pltpu.sync_copy(paddst_t, ldst0)
  pltpu.sync_copy(paddst_t, ldst1)
  pltpu.sync_copy(zeros_hbm, hist_s)
  pltpu.sync_copy(zeros_hbm, hist_d)

  def body(g, carry):
    cnt0, cnt1 = carry
    for j in range(CH // 16):
      sl = pl.ds(j * 16, 16)
      sv = src_v.at[g][sl]
      dv = dst_v.at[g][sl]
      dup_s, last_s = plsc.scan_count(sv)
      plsc.addupdate_scatter(hist_s, [sv], dup_s, mask=last_s)
      dup_d, last_d = plsc.scan_count(dv)
      plsc.addupdate_scatter(hist_d, [dv], dup_d, mask=last_d)
      m0 = dv < NH
      ones0 = jnp.where(m0, 1, 0).astype(jnp.int32)
      pos0 = cnt0 + plsc.cumsum(ones0) - 1
      plsc.store_scatter(lsrc0, [pos0], sv, mask=m0)
      plsc.store_scatter(ldst0, [pos0], dv, mask=m0)
      cnt0 = cnt0 + plsc.all_reduce_population_count(m0)
      m1 = jnp.logical_not(m0)
      ones1 = jnp.where(m1, 1, 0).astype(jnp.int32)
      pos1 = cnt1 + plsc.cumsum(ones1) - 1
      plsc.store_scatter(lsrc1, [pos1], sv, mask=m1)
      plsc.store_scatter(ldst1, [pos1], dv, mask=m1)
      cnt1 = cnt1 + plsc.all_reduce_population_count(m1)
    return cnt0, cnt1

  zero16 = jnp.zeros((16,), jnp.int32)
  cnt0, cnt1 = lax.fori_loop(0, GPT_DEG, body, (zero16, zero16))
  lane = lax.iota(jnp.int32, 16)
  cnt_v[...] = jnp.where(lane == 0, cnt0, jnp.where(lane == 1, cnt1, 0))
  pltpu.sync_copy(cnt_v, rcnt.at[wid])
  pltpu.sync_copy(lsrc0.at[pl.ds(0, LSEG * CH)], rsrc.at[wid, 0])
  pltpu.sync_copy(lsrc1.at[pl.ds(0, LSEG * CH)], rsrc.at[wid, 1])
  pltpu.sync_copy(ldst0.at[pl.ds(0, LSEG * CH)], rdst.at[wid, 0])
  pltpu.sync_copy(ldst1.at[pl.ds(0, LSEG * CH)], rdst.at[wid, 1])
  pltpu.sync_copy(hist_s, out_hs.at[wid])
  pltpu.sync_copy(hist_d, out_hd.at[wid])


@functools.cache
def _make_spmm():
  mesh = plsc.VectorSubcoreMesh(
      core_axis_name="c", subcore_axis_name="s",
      num_cores=NC, num_subcores=NS)
  return functools.partial(
      pl.kernel,
      out_type=jax.ShapeDtypeStruct((NPAD, D), jnp.float32),
      mesh=mesh,
      scratch_types=[
          pltpu.VMEM((LSEG, CH), jnp.int32),
          pltpu.VMEM((LSEG, CH), jnp.int32),
          pltpu.VMEM((NB, CH), jnp.int32),
          pltpu.VMEM((NB, CH, D), jnp.float32),
          pltpu.VMEM((16,), jnp.int32),
          pltpu.VMEM_SHARED((NACC, D), jnp.float32),
          pltpu.SemaphoreType.DMA,
          pltpu.SemaphoreType.DMA,
      ],
  )(_spmm_body)


def _spmm_body(tbl, rsrc, rdst, rcnt, zeros_hbm, out,
               src_v, dst_v, dloc_v, rows, cnt_v, acc, gsem0, gsem1):
  c = lax.axis_index("c")
  s = lax.axis_index("s")
  base = c * NH
  pltpu.sync_copy(zeros_hbm, acc.at[pl.ds(s * ZR, ZR)])
  plsc.subcore_barrier()
  gsems = (gsem0, gsem1)

  def localize(g, b):
    for j in range(CH // 16):
      sl = pl.ds(j * 16, 16)
      dv = dst_v.at[g][sl]
      lv = dv - base
      valid = (lv >= 0) & (lv < NH)
      dump = NH + j * 16 + lax.iota(jnp.int32, 16)
      dloc_v.at[b][sl] = jnp.where(valid, lv, dump)

  def run_segment(k):
    w = 2 * s + k
    pltpu.sync_copy(rsrc.at[w, c], src_v)
    pltpu.sync_copy(rdst.at[w, c], dst_v)
    pltpu.sync_copy(rcnt.at[w], cnt_v)
    cv = cnt_v[...]
    cnt = jnp.where(c == 0, cv[0], cv[1])
    nchunk = lax.div(cnt + (CH - 1), CH)
    nt = jnp.maximum(lax.div(nchunk + (NB - 1), NB), 1)

    def gather(g, b):
      pltpu.async_copy(tbl.at[src_v.at[g]], rows.at[b], gsems[b % 2])

    gather(0, 0)
    gather(1, 1)

    def body(g4, carry):
      for b in range(NB):
        g = g4 * NB + b
        pltpu.make_async_copy(tbl.at[src_v.at[g]], rows.at[b],
                              gsems[b % 2]).wait()

        @pl.when(g + 2 < nt * NB)
        def _prefetch():
          gather(g + 2, (b + 2) % NB)

        localize(g, b)
        pltpu.sync_copy(rows.at[b], acc.at[dloc_v.at[b]], add=True)
      return carry

    lax.fori_loop(0, nt, body, 0)

  run_segment(0)
  run_segment(1)
  plsc.subcore_barrier()
  pltpu.sync_copy(acc.at[pl.ds(s * WR, WR)],
                  out.at[pl.ds(base + s * WR, WR)])


def _norm_from(hist):
  deg = jnp.sum(hist, axis=0).astype(jnp.float32)
  return jnp.where(deg > 0.0, lax.rsqrt(jnp.maximum(deg, 1.0)), 0.0)


def _tc1_body(x_ref, degs_ref, w_ref, out_ref):
  ns = _norm_from(degs_ref[...])
  xs = x_ref[...] * ns[:, None]
  out_ref[...] = jnp.dot(xs, w_ref[...], preferred_element_type=jnp.float32)


def _tc2_body(agg_ref, degs_ref, degd_ref, b_ref, w_ref, out_ref):
  nd = _norm_from(degd_ref[...])
  h = jnp.maximum(agg_ref[...] * nd[:, None] + b_ref[...], 0.0)
  ns = _norm_from(degs_ref[...])
  out_ref[...] = jnp.dot(h * ns[:, None], w_ref[...],
                         preferred_element_type=jnp.float32)


def _tc3_body(agg_ref, degd_ref, b_ref, wc_ref, bc_ref, out_ref):
  nd = _norm_from(degd_ref[...])
  h = jnp.maximum(agg_ref[...] * nd[:, None] + b_ref[...], 0.0)
  hg = jnp.sum(h[:N, :], axis=0, keepdims=True) * (1.0 / N)
  out_ref[...] = jnp.dot(hg, wc_ref[...],
                         preferred_element_type=jnp.float32) + bc_ref[...]


def kernel(x, edge_index, W1, b1, W2, b2, Wc, bc):
  f32 = jnp.float32
  src = edge_index[0].astype(jnp.int32)
  dst = edge_index[1].astype(jnp.int32)
  npad_rows = NPAD - N
  padv = N + (jnp.arange(EPAD - E, dtype=jnp.int32) % npad_rows)
  src_flat = jnp.concatenate([src, padv])
  dst_flat = jnp.concatenate([dst, padv])
  srcp32 = src_flat.reshape(NW, GPT_DEG, CH)
  dstp32 = dst_flat.reshape(NW, GPT_DEG, CH)
  xp = jnp.concatenate([x, jnp.zeros((npad_rows, D), f32)], axis=0)

  zi = jnp.zeros((NPAD,), jnp.int32)
  zacc = jnp.zeros((ZR, D), f32)
  padsrc_t = N + (jnp.arange(LCAP, dtype=jnp.int32) % npad_rows)
  paddst_t = jnp.full((LCAP,), 2 * NPAD, jnp.int32)

  rsrc, rdst, rcnt, deg_s, deg_d = _make_prep()(
      srcp32, dstp32, padsrc_t, paddst_t, zi)
  rsrc4 = rsrc.reshape(NW, NC, LSEG, CH)
  rdst4 = rdst.reshape(NW, NC, LSEG, CH)

  hw1 = pl.pallas_call(
      _tc1_body,
      out_shape=jax.ShapeDtypeStruct((NPAD, D), f32),
  )(xp, deg_s, W1)

  agg1 = _make_spmm()(hw1, rsrc4, rdst4, rcnt, zacc)

  hw2 = pl.pallas_call(
      _tc2_body,
      out_shape=jax.ShapeDtypeStruct((NPAD, D), f32),
  )(agg1, deg_s, deg_d, b1.reshape(1, D), W2)

  agg2 = _make_spmm()(hw2, rsrc4, rdst4, rcnt, zacc)

  wcp = jnp.zeros((D, D), f32).at[:, :Wc.shape[1]].set(Wc)
  bcp = jnp.zeros((1, D), f32).at[0, :bc.shape[0]].set(bc)
  outp = pl.pallas_call(
      _tc3_body,
      out_shape=jax.ShapeDtypeStruct((1, D), f32),
  )(agg2, deg_d, b2.reshape(1, D), wcp, bcp)
  return outp[:, :Wc.shape[1]]


# speedup vs baseline: 12.3168x; 1.0173x over previous
import functools

import jax
import jax.numpy as jnp
from jax import lax
from jax.experimental import pallas as pl
from jax.experimental.pallas import tpu as pltpu
from jax.experimental.pallas import tpu_sc as plsc

N = 10000
NPAD = 10240
D = 128
DH = 64
E = 320000
NC, NS = 2, 16
NW = NC * NS
CH = 128
GPT_DEG = 80
EPAD = NW * GPT_DEG * CH
RPT = NPAD // NS


NB = 4
NH = NPAD // NC
NDUMP = 128
NACC = NH + NDUMP
ZR = NACC // NS
WR = NH // NS
LSEG = GPT_DEG
LCAP = LSEG * CH + 16


@functools.cache
def _make_prep():
  mesh = plsc.VectorSubcoreMesh(
      core_axis_name="c", subcore_axis_name="s",
      num_cores=NC, num_subcores=NS)
  return functools.partial(
      pl.kernel,
      out_type=(jax.ShapeDtypeStruct((NW, NC, LSEG * CH), jnp.int32),
                jax.ShapeDtypeStruct((NW, NC, LSEG * CH), jnp.int32),
                jax.ShapeDtypeStruct((NW, 16), jnp.int32),
                jax.ShapeDtypeStruct((NW, NPAD), jnp.int32),
                jax.ShapeDtypeStruct((NW, NPAD), jnp.int32)),
      mesh=mesh,
      scratch_types=[
          pltpu.VMEM((GPT_DEG, CH), jnp.int32),
          pltpu.VMEM((GPT_DEG, CH), jnp.int32),
          pltpu.VMEM((LCAP,), jnp.int32),
          pltpu.VMEM((LCAP,), jnp.int32),
          pltpu.VMEM((LCAP,), jnp.int32),
          pltpu.VMEM((LCAP,), jnp.int32),
          pltpu.VMEM((16,), jnp.int32),
          pltpu.VMEM((NPAD,), jnp.int32),
          pltpu.VMEM((NPAD,), jnp.int32),
      ],
      compiler_params=pltpu.CompilerParams(needs_layout_passes=False),
  )(_prep_body)


def _prep_body(srcp, dstp, padsrc_t, paddst_t, zeros_hbm,
               rsrc, rdst, rcnt, out_hs, out_hd,
               src_v, dst_v, lsrc0, lsrc1, ldst0, ldst1, cnt_v,
               hist_s, hist_d):
  c = lax.axis_index("c")
  s = lax.axis_index("s")
  wid = c * NS + s
  pltpu.sync_copy(srcp.at[wid], src_v)
  pltpu.sync_copy(dstp.at[wid], dst_v)
  pltpu.sync_copy(padsrc_t, lsrc0)
  pltpu.sync_copy(padsrc_t, lsrc1)
  pltpu.sync_copy(paddst_t, ldst0)
  pltpu.sync_copy(paddst_t, ldst1)
  pltpu.sync_copy(zeros_hbm, hist_s)
  pltpu.sync_copy(zeros_hbm, hist_d)

  def body(g, carry):
    cnt0, cnt1 = carry
    for j in range(CH // 16):
      sl = pl.ds(j * 16, 16)
      sv = src_v.at[g][sl]
      dv = dst_v.at[g][sl]
      dup_s, last_s = plsc.scan_count(sv)
      plsc.addupdate_scatter(hist_s, [sv], dup_s, mask=last_s)
      dup_d, last_d = plsc.scan_count(dv)
      plsc.addupdate_scatter(hist_d, [dv], dup_d, mask=last_d)
      m0 = dv < NH
      ones0 = jnp.where(m0, 1, 0).astype(jnp.int32)
      pos0 = cnt0 + plsc.cumsum(ones0) - 1
      plsc.store_scatter(lsrc0, [pos0], sv, mask=m0)
      plsc.store_scatter(ldst0, [pos0], dv, mask=m0)
      cnt0 = cnt0 + plsc.all_reduce_population_count(m0)
      m1 = jnp.logical_not(m0)
      ones1 = jnp.where(m1, 1, 0).astype(jnp.int32)
      pos1 = cnt1 + plsc.cumsum(ones1) - 1
      plsc.store_scatter(lsrc1, [pos1], sv, mask=m1)
      plsc.store_scatter(ldst1, [pos1], dv - NH, mask=m1)
      cnt1 = cnt1 + plsc.all_reduce_population_count(m1)
    return cnt0, cnt1

  zero16 = jnp.zeros((16,), jnp.int32)
  cnt0, cnt1 = lax.fori_loop(0, GPT_DEG, body, (zero16, zero16))
  lane = lax.iota(jnp.int32, 16)
  cnt_v[...] = jnp.where(lane == 0, cnt0, jnp.where(lane == 1, cnt1, 0))
  pltpu.sync_copy(cnt_v, rcnt.at[wid])
  pltpu.sync_copy(lsrc0.at[pl.ds(0, LSEG * CH)], rsrc.at[wid, 0])
  pltpu.sync_copy(lsrc1.at[pl.ds(0, LSEG * CH)], rsrc.at[wid, 1])
  pltpu.sync_copy(ldst0.at[pl.ds(0, LSEG * CH)], rdst.at[wid, 0])
  pltpu.sync_copy(ldst1.at[pl.ds(0, LSEG * CH)], rdst.at[wid, 1])
  pltpu.sync_copy(hist_s, out_hs.at[wid])
  pltpu.sync_copy(hist_d, out_hd.at[wid])


@functools.cache
def _make_spmm():
  mesh = plsc.VectorSubcoreMesh(
      core_axis_name="c", subcore_axis_name="s",
      num_cores=NC, num_subcores=NS)
  return functools.partial(
      pl.kernel,
      out_type=jax.ShapeDtypeStruct((NPAD, D), jnp.float32),
      mesh=mesh,
      scratch_types=[
          pltpu.VMEM((LSEG, CH), jnp.int32),
          pltpu.VMEM((LSEG, CH), jnp.int32),
          pltpu.VMEM((NB, CH, D), jnp.float32),
          pltpu.VMEM((16,), jnp.int32),
          pltpu.VMEM_SHARED((NACC, D), jnp.float32),
          pltpu.SemaphoreType.DMA,
          pltpu.SemaphoreType.DMA,
      ],
  )(_spmm_body)


def _spmm_body(tbl, rsrc, rdst, rcnt, zeros_hbm, out,
               src_v, dst_v, rows, cnt_v, acc, gsem0, gsem1):
  c = lax.axis_index("c")
  s = lax.axis_index("s")
  base = c * NH
  pltpu.sync_copy(zeros_hbm, acc.at[pl.ds(s * ZR, ZR)])
  plsc.subcore_barrier()
  gsems = (gsem0, gsem1)

  def run_segment(k):
    w = 2 * s + k
    pltpu.sync_copy(rsrc.at[w, c], src_v)
    pltpu.sync_copy(rdst.at[w, c], dst_v)
    pltpu.sync_copy(rcnt.at[w], cnt_v)
    cv = cnt_v[...]
    cnt = jnp.where(c == 0, cv[0], cv[1])
    nchunk = lax.div(cnt + (CH - 1), CH)
    nt = jnp.maximum(lax.div(nchunk + (NB - 1), NB), 1)

    def gather(g, b):
      pltpu.async_copy(tbl.at[src_v.at[g]], rows.at[b], gsems[b % 2])

    gather(0, 0)
    gather(1, 1)

    def body(g4, carry):
      for b in range(NB):
        g = g4 * NB + b
        pltpu.make_async_copy(tbl.at[src_v.at[g]], rows.at[b],
                              gsems[b % 2]).wait()

        @pl.when(g + 2 < nt * NB)
        def _prefetch():
          gather(g + 2, (b + 2) % NB)

        pltpu.sync_copy(rows.at[b], acc.at[dst_v.at[g]], add=True)
      return carry

    lax.fori_loop(0, nt, body, 0)

  run_segment(0)
  run_segment(1)
  plsc.subcore_barrier()
  pltpu.sync_copy(acc.at[pl.ds(s * WR, WR)],
                  out.at[pl.ds(base + s * WR, WR)])


def _norm_from(hist):
  deg = jnp.sum(hist, axis=0).astype(jnp.float32)
  return jnp.where(deg > 0.0, lax.rsqrt(jnp.maximum(deg, 1.0)), 0.0)


def _tc1_body(x_ref, degs_ref, w_ref, out_ref):
  ns = _norm_from(degs_ref[...])
  xs = x_ref[...] * ns[:, None]
  out_ref[...] = jnp.dot(xs, w_ref[...], preferred_element_type=jnp.float32)


def _tc2_body(agg_ref, degs_ref, degd_ref, b_ref, w_ref, out_ref):
  nd = _norm_from(degd_ref[...])
  h = jnp.maximum(agg_ref[...] * nd[:, None] + b_ref[...], 0.0)
  ns = _norm_from(degs_ref[...])
  out_ref[...] = jnp.dot(h * ns[:, None], w_ref[...],
                         preferred_element_type=jnp.float32)


def _tc3_body(agg_ref, degd_ref, b_ref, wc_ref, bc_ref, out_ref):
  nd = _norm_from(degd_ref[...])
  h = jnp.maximum(agg_ref[...] * nd[:, None] + b_ref[...], 0.0)
  hg = jnp.sum(h[:N, :], axis=0, keepdims=True) * (1.0 / N)
  out_ref[...] = jnp.dot(hg, wc_ref[...],
                         preferred_element_type=jnp.float32) + bc_ref[...]


def kernel(x, edge_index, W1, b1, W2, b2, Wc, bc):
  f32 = jnp.float32
  src = edge_index[0].astype(jnp.int32)
  dst = edge_index[1].astype(jnp.int32)
  npad_rows = NPAD - N
  padv = N + (jnp.arange(EPAD - E, dtype=jnp.int32) % npad_rows)
  src_flat = jnp.concatenate([src, padv])
  dst_flat = jnp.concatenate([dst, padv])
  srcp32 = src_flat.reshape(NW, GPT_DEG, CH)
  dstp32 = dst_flat.reshape(NW, GPT_DEG, CH)
  xp = jnp.concatenate([x, jnp.zeros((npad_rows, D), f32)], axis=0)

  zi = jnp.zeros((NPAD,), jnp.int32)
  zacc = jnp.zeros((ZR, D), f32)
  padsrc_t = N + (jnp.arange(LCAP, dtype=jnp.int32) % npad_rows)
  paddst_t = NH + (jnp.arange(LCAP, dtype=jnp.int32) % NDUMP)

  rsrc, rdst, rcnt, deg_s, deg_d = _make_prep()(
      srcp32, dstp32, padsrc_t, paddst_t, zi)
  rsrc4 = rsrc.reshape(NW, NC, LSEG, CH)
  rdst4 = rdst.reshape(NW, NC, LSEG, CH)

  hw1 = pl.pallas_call(
      _tc1_body,
      out_shape=jax.ShapeDtypeStruct((NPAD, D), f32),
  )(xp, deg_s, W1)

  agg1 = _make_spmm()(hw1, rsrc4, rdst4, rcnt, zacc)

  hw2 = pl.pallas_call(
      _tc2_body,
      out_shape=jax.ShapeDtypeStruct((NPAD, D), f32),
  )(agg1, deg_s, deg_d, b1.reshape(1, D), W2)

  agg2 = _make_spmm()(hw2, rsrc4, rdst4, rcnt, zacc)

  wcp = jnp.zeros((D, D), f32).at[:, :Wc.shape[1]].set(Wc)
  bcp = jnp.zeros((1, D), f32).at[0, :bc.shape[0]].set(bc)
  outp = pl.pallas_call(
      _tc3_body,
      out_shape=jax.ShapeDtypeStruct((1, D), f32),
  )(agg2, deg_d, b2.reshape(1, D), wcp, bcp)
  return outp[:, :Wc.shape[1]]
